# baseline (speedup 1.0000x reference)
import functools

import jax
import jax.numpy as jnp
import numpy as np
from jax import lax
from jax.experimental import pallas as pl
from jax.experimental.pallas import tpu as pltpu
from jax.experimental.pallas import tpu_sc as plsc

SEQLEN = 2048
D_MODEL = 768
BATCH = 4
SCALE = float(np.sqrt(float(D_MODEL)))

NC, NS, L = 2, 16, 16
NW = NC * NS
SEQ_PER_W = SEQLEN // NW
CK = 32
CH_PER_B = SEQ_PER_W // CK
NCH = BATCH * CH_PER_B


def _position_encoding(seqlen, d_model, times=10000):
    pos = np.arange(seqlen)[:, np.newaxis].astype(np.float64)
    depths = np.arange(d_model)[np.newaxis, :].astype(np.float64)
    depths = 2 * (depths // 2) / d_model
    angle_rates = 1.0 / times ** depths
    angle_rads = pos * angle_rates
    pe = np.zeros((seqlen, d_model), dtype=np.float64)
    pe[:, 0::2] = np.sin(angle_rads)[:, 0::2]
    pe[:, 1::2] = np.cos(angle_rads)[:, 1::2]
    return pe.astype(np.float32)


_PE = _position_encoding(SEQLEN, D_MODEL).reshape(NW, SEQ_PER_W, D_MODEL)

_mesh = plsc.VectorSubcoreMesh(core_axis_name="c", subcore_axis_name="s")


@functools.partial(
    pl.kernel,
    mesh=_mesh,
    out_type=jax.ShapeDtypeStruct((BATCH * SEQLEN, D_MODEL), jnp.float32),
    scratch_types=[
        pltpu.VMEM((NCH, CK), jnp.int32),
        pltpu.VMEM((SEQ_PER_W, D_MODEL), jnp.float32),
        pltpu.VMEM((CK, D_MODEL), jnp.float32),
        pltpu.VMEM((CK, D_MODEL), jnp.float32),
        pltpu.SemaphoreType.DMA,
        pltpu.SemaphoreType.DMA,
        pltpu.SemaphoreType.DMA,
        pltpu.SemaphoreType.DMA,
    ],
)
def _emb(x_hbm, pe_hbm, table_hbm, out_hbm,
         idx_v, pe_v, buf0, buf1, g0, g1, o0, o1):
    wid = lax.axis_index("s") * NC + lax.axis_index("c")
    pltpu.sync_copy(x_hbm.at[wid], idx_v)
    bufs, gsems, osems = (buf0, buf1), (g0, g1), (o0, o1)

    pe_cp = pltpu.async_copy(pe_hbm.at[wid], pe_v, o0)
    gathers = [pltpu.async_copy(table_hbm.at[idx_v.at[0]], buf0, g0)]
    scatters = [None] * NCH
    pe_cp.wait()

    for k in range(NCH):
        b, c = divmod(k, CH_PER_B)
        buf = bufs[k % 2]
        if k + 1 < NCH:
            if k >= 1:
                scatters[k - 1].wait()
            gathers.append(pltpu.async_copy(
                table_hbm.at[idx_v.at[k + 1]], bufs[(k + 1) % 2],
                gsems[(k + 1) % 2]))
        gathers[k].wait()

        def row_body(i, _, buf=buf, c=c):
            for j in range(D_MODEL // L):
                sl = pl.ds(j * L, L)
                buf[i, sl] = buf[i, sl] * SCALE + pe_v[c * CK + i, sl]
            return _

        lax.fori_loop(0, CK, row_body, None)
        out_off = b * SEQLEN + wid * SEQ_PER_W + c * CK
        scatters[k] = pltpu.async_copy(
            buf, out_hbm.at[pl.ds(out_off, CK)], osems[k % 2])

    scatters[NCH - 2].wait()
    scatters[NCH - 1].wait()


def kernel(x, table):
    idx = (x.astype(jnp.int32)
           .reshape(BATCH, NW, CH_PER_B, CK)
           .transpose(1, 0, 2, 3)
           .reshape(NW, NCH, CK))
    out = _emb(idx, _PE, table)
    return out.reshape(BATCH, SEQLEN, D_MODEL)
